# baseline (speedup 1.0000x reference)
import functools

import jax
import jax.numpy as jnp
import numpy as np
from jax import lax
from jax.experimental import pallas as pl
from jax.experimental.pallas import tpu as pltpu
from jax.experimental.pallas import tpu_sc as plsc

N = 10000
E = 320000
D = 128
H = 128
G = 64
NC = 2

NCORES = 2
NSUB = 16
NW = NCORES * NSUB
EPW = E // NW
CH = 256
NCH = 40
EPAD = NCH * CH - EPW
NDUMMY = 8
RPT = 640

_sc_mesh = plsc.VectorSubcoreMesh(core_axis_name="c", subcore_axis_name="s")


@functools.partial(
    pl.kernel,
    out_type=jax.ShapeDtypeStruct((2 * N, H), jnp.float32),
    mesh=_sc_mesh,
    scratch_types=[
        pltpu.VMEM((CH,), jnp.int32),
        pltpu.VMEM((CH,), jnp.int32),
        pltpu.VMEM((CH, H), jnp.float32),
        pltpu.SemaphoreType.DMA,
        pltpu.VMEM_SHARED((N + NDUMMY, H), jnp.float32),
    ],
)
def _sc_agg(h_hbm, src_hbm, dst_hbm, out_hbm,
            sidx, didx, rows0, gsem, agg_sh):
    cid = lax.axis_index("c")
    sid = lax.axis_index("s")
    wid = cid * NSUB + sid

    ZB = 128
    def _zrow(r, carry):
        for c8 in range(H // 16):
            rows0[r, pl.ds(c8 * 16, 16)] = jnp.zeros((16,), jnp.float32)
        return carry
    lax.fori_loop(0, ZB, _zrow, 0)
    row0 = sid * RPT

    @pl.when(sid < NSUB - 1)
    def _():
        for t in range(RPT // ZB):
            pltpu.sync_copy(rows0.at[pl.ds(0, ZB)],
                            agg_sh.at[pl.ds(row0 + t * ZB, ZB)])

    @pl.when(sid == NSUB - 1)
    def _():
        for t in range(3):
            pltpu.sync_copy(rows0.at[pl.ds(0, ZB)],
                            agg_sh.at[pl.ds(row0 + t * ZB, ZB)])
        last = N + NDUMMY - (NSUB - 1) * RPT - 3 * ZB
        pltpu.sync_copy(rows0.at[pl.ds(0, last)],
                        agg_sh.at[pl.ds(row0 + 3 * ZB, last)])
    plsc.subcore_barrier()

    def _chunk(j, carry):
        pltpu.sync_copy(src_hbm.at[wid, j], sidx)
        pltpu.sync_copy(dst_hbm.at[wid, j], didx)
        pltpu.async_copy(h_hbm.at[sidx], rows0, gsem).wait()
        pltpu.sync_copy(rows0, agg_sh.at[didx], add=True)
        return carry
    lax.fori_loop(0, NCH, _chunk, 0)

    plsc.subcore_barrier()

    @pl.when(sid < NSUB - 1)
    def _():
        pltpu.sync_copy(agg_sh.at[pl.ds(row0, RPT)],
                        out_hbm.at[pl.ds(cid * N + row0, RPT)])

    @pl.when(sid == NSUB - 1)
    def _():
        pltpu.sync_copy(agg_sh.at[pl.ds(row0, N - (NSUB - 1) * RPT)],
                        out_hbm.at[pl.ds(cid * N + row0, N - (NSUB - 1) * RPT)])


BR = 1000
NBLK = N // BR


def _mlp_body(h_ref, a0_ref, a1_ref, w1_ref, b1_ref, w2_ref, b2_ref, o_ref):
    z = h_ref[...] + a0_ref[...] + a1_ref[...]
    t = jnp.maximum(
        jnp.dot(z, w1_ref[...], preferred_element_type=jnp.float32) + b1_ref[...],
        0.0)
    o_ref[...] = jnp.maximum(
        jnp.dot(t, w2_ref[...], preferred_element_type=jnp.float32) + b2_ref[...],
        0.0)


_mlp = pl.pallas_call(
    _mlp_body,
    grid=(NBLK,),
    in_specs=[
        pl.BlockSpec((BR, H), lambda i: (i, 0)),
        pl.BlockSpec((BR, H), lambda i: (i, 0)),
        pl.BlockSpec((BR, H), lambda i: (NBLK + i, 0)),
        pl.BlockSpec((H, H), lambda i: (0, 0)),
        pl.BlockSpec((1, H), lambda i: (0, 0)),
        pl.BlockSpec((H, H), lambda i: (0, 0)),
        pl.BlockSpec((1, H), lambda i: (0, 0)),
    ],
    out_specs=pl.BlockSpec((BR, H), lambda i: (i, 0)),
    out_shape=jax.ShapeDtypeStruct((N, H), jnp.float32),
)

_BN_SCALE = float(1.0 / np.sqrt(1.0 + 1e-5))


def _pool_cls_body(b_ref, h1_ref, h2_ref, h3_ref, cw1_ref, cb1_ref,
                   g_ref, be_ref, cw2_ref, cb2_ref, o_ref, acc_ref):
    i = pl.program_id(0)

    @pl.when(i == 0)
    def _():
        acc_ref[...] = jnp.zeros_like(acc_ref)

    oh = (b_ref[...] == lax.broadcasted_iota(jnp.int32, (1, G), 1)
          ).astype(jnp.float32)
    hcat = jnp.concatenate([h1_ref[...], h2_ref[...], h3_ref[...]], axis=1)
    acc_ref[...] += jnp.dot(oh.T, hcat, preferred_element_type=jnp.float32)

    @pl.when(i == pl.num_programs(0) - 1)
    def _():
        z = jnp.dot(acc_ref[...], cw1_ref[...],
                    preferred_element_type=jnp.float32) + cb1_ref[...]
        z = z * _BN_SCALE * g_ref[...] + be_ref[...]
        z = jnp.maximum(z, 0.0)
        o_ref[...] = jnp.dot(z, cw2_ref[...],
                             preferred_element_type=jnp.float32) + cb2_ref[...]


_pool_cls = pl.pallas_call(
    _pool_cls_body,
    grid=(NBLK,),
    in_specs=[
        pl.BlockSpec((BR, 1), lambda i: (i, 0)),
        pl.BlockSpec((BR, H), lambda i: (i, 0)),
        pl.BlockSpec((BR, H), lambda i: (i, 0)),
        pl.BlockSpec((BR, H), lambda i: (i, 0)),
        pl.BlockSpec((3 * H, 2 * H), lambda i: (0, 0)),
        pl.BlockSpec((1, 2 * H), lambda i: (0, 0)),
        pl.BlockSpec((1, 2 * H), lambda i: (0, 0)),
        pl.BlockSpec((1, 2 * H), lambda i: (0, 0)),
        pl.BlockSpec((2 * H, 128), lambda i: (0, 0)),
        pl.BlockSpec((1, 128), lambda i: (0, 0)),
    ],
    out_specs=pl.BlockSpec((G, 128), lambda i: (0, 0)),
    out_shape=jax.ShapeDtypeStruct((G, 128), jnp.float32),
    scratch_shapes=[pltpu.VMEM((G, 3 * H), jnp.float32)],
)


def kernel(x, edge_index, batch, W1_0, b1_0, W2_0, b2_0, W1_1, b1_1, W2_1,
           b2_1, W1_2, b1_2, W2_2, b2_2, cW1, cb1, bn_gamma, bn_beta, cW2,
           cb2):
    src_p = jnp.pad(edge_index[0].reshape(NW, EPW),
                    ((0, 0), (0, EPAD))).reshape(NW, NCH, CH)
    dst_p = jnp.pad(edge_index[1].reshape(NW, EPW),
                    ((0, 0), (0, EPAD)),
                    constant_values=N).reshape(NW, NCH, CH)
    params = [(W1_0, b1_0, W2_0, b2_0), (W1_1, b1_1, W2_1, b2_1),
              (W1_2, b1_2, W2_2, b2_2)]

    h = x
    hs = []
    for (W1, b1, W2, b2) in params:
        agg = _sc_agg(h, src_p, dst_p)
        h = _mlp(h, agg, agg, W1, b1.reshape(1, H), W2, b2.reshape(1, H))
        hs.append(h)

    cW2p = jnp.zeros((2 * H, 128), jnp.float32).at[:, :NC].set(cW2)
    cb2p = jnp.zeros((1, 128), jnp.float32).at[0, :NC].set(cb2)
    out = _pool_cls(batch.reshape(N, 1), hs[0], hs[1], hs[2], cW1,
                    cb1.reshape(1, 2 * H), bn_gamma.reshape(1, 2 * H),
                    bn_beta.reshape(1, 2 * H), cW2p, cb2p)
    return out[:, :NC]


# speedup vs baseline: 1.8364x; 1.8364x over previous
import functools

import jax
import jax.numpy as jnp
import numpy as np
from jax import lax
from jax.experimental import pallas as pl
from jax.experimental.pallas import tpu as pltpu
from jax.experimental.pallas import tpu_sc as plsc

N = 10000
E = 320000
D = 128
H = 128
G = 64
NC = 2

NCORES = 2
NSUB = 16
NW = NCORES * NSUB
EPW = E // NW
CH = 128
NFULL = EPW // CH
TAIL = EPW - NFULL * CH
RPT = 640

_sc_mesh = plsc.VectorSubcoreMesh(core_axis_name="c", subcore_axis_name="s")


@functools.partial(
    pl.kernel,
    out_type=jax.ShapeDtypeStruct((2 * N, H), jnp.float32),
    mesh=_sc_mesh,
    scratch_types=[
        pltpu.VMEM((CH,), jnp.int32),
        pltpu.VMEM((CH,), jnp.int32),
        pltpu.VMEM((CH, H), jnp.float32),
        pltpu.VMEM((TAIL,), jnp.int32),
        pltpu.VMEM((TAIL,), jnp.int32),
        pltpu.VMEM((TAIL, H), jnp.float32),
        pltpu.SemaphoreType.DMA,
        pltpu.VMEM_SHARED((N, H), jnp.float32),
    ],
)
def _sc_agg(h_hbm, src_hbm, dst_hbm, out_hbm,
            sidx, didx, rows0, sidx2, didx2, rows2, gsem, agg_sh):
    cid = lax.axis_index("c")
    sid = lax.axis_index("s")
    wid = cid * NSUB + sid
    base = wid * EPW

    ZB = 128
    def _zrow(r, carry):
        for c8 in range(H // 16):
            rows0[r, pl.ds(c8 * 16, 16)] = jnp.zeros((16,), jnp.float32)
        return carry
    lax.fori_loop(0, ZB, _zrow, 0)
    row0 = sid * RPT

    @pl.when(sid < NSUB - 1)
    def _():
        for t in range(RPT // ZB):
            pltpu.sync_copy(rows0.at[pl.ds(0, ZB)],
                            agg_sh.at[pl.ds(row0 + t * ZB, ZB)])

    @pl.when(sid == NSUB - 1)
    def _():
        for t in range(3):
            pltpu.sync_copy(rows0.at[pl.ds(0, ZB)],
                            agg_sh.at[pl.ds(row0 + t * ZB, ZB)])
        last = N - (NSUB - 1) * RPT - 3 * ZB
        pltpu.sync_copy(rows0.at[pl.ds(0, last)],
                        agg_sh.at[pl.ds(row0 + 3 * ZB, last)])
    plsc.subcore_barrier()

    def _chunk(j, carry):
        off = base + j * CH
        pltpu.sync_copy(src_hbm.at[pl.ds(off, CH)], sidx)
        pltpu.sync_copy(dst_hbm.at[pl.ds(off, CH)], didx)
        pltpu.async_copy(h_hbm.at[sidx], rows0, gsem).wait()
        pltpu.sync_copy(rows0, agg_sh.at[didx], add=True)
        return carry
    lax.fori_loop(0, NFULL, _chunk, 0)

    off = base + NFULL * CH
    pltpu.sync_copy(src_hbm.at[pl.ds(off, TAIL)], sidx2)
    pltpu.sync_copy(dst_hbm.at[pl.ds(off, TAIL)], didx2)
    pltpu.async_copy(h_hbm.at[sidx2], rows2, gsem).wait()
    pltpu.sync_copy(rows2, agg_sh.at[didx2], add=True)

    plsc.subcore_barrier()

    @pl.when(sid < NSUB - 1)
    def _():
        pltpu.sync_copy(agg_sh.at[pl.ds(row0, RPT)],
                        out_hbm.at[pl.ds(cid * N + row0, RPT)])

    @pl.when(sid == NSUB - 1)
    def _():
        pltpu.sync_copy(agg_sh.at[pl.ds(row0, N - (NSUB - 1) * RPT)],
                        out_hbm.at[pl.ds(cid * N + row0, N - (NSUB - 1) * RPT)])


BR = 1000
NBLK = N // BR


def _mlp_body(h_ref, a0_ref, a1_ref, w1_ref, b1_ref, w2_ref, b2_ref, o_ref):
    z = h_ref[...] + a0_ref[...] + a1_ref[...]
    t = jnp.maximum(
        jnp.dot(z, w1_ref[...], preferred_element_type=jnp.float32) + b1_ref[...],
        0.0)
    o_ref[...] = jnp.maximum(
        jnp.dot(t, w2_ref[...], preferred_element_type=jnp.float32) + b2_ref[...],
        0.0)


_mlp = pl.pallas_call(
    _mlp_body,
    grid=(NBLK,),
    in_specs=[
        pl.BlockSpec((BR, H), lambda i: (i, 0)),
        pl.BlockSpec((BR, H), lambda i: (i, 0)),
        pl.BlockSpec((BR, H), lambda i: (NBLK + i, 0)),
        pl.BlockSpec((H, H), lambda i: (0, 0)),
        pl.BlockSpec((1, H), lambda i: (0, 0)),
        pl.BlockSpec((H, H), lambda i: (0, 0)),
        pl.BlockSpec((1, H), lambda i: (0, 0)),
    ],
    out_specs=pl.BlockSpec((BR, H), lambda i: (i, 0)),
    out_shape=jax.ShapeDtypeStruct((N, H), jnp.float32),
)

_BN_SCALE = float(1.0 / np.sqrt(1.0 + 1e-5))


def _pool_cls_body(b_ref, h1_ref, h2_ref, h3_ref, cw1_ref, cb1_ref,
                   g_ref, be_ref, cw2_ref, cb2_ref, o_ref, acc_ref):
    i = pl.program_id(0)

    @pl.when(i == 0)
    def _():
        acc_ref[...] = jnp.zeros_like(acc_ref)

    oh = (b_ref[...] == lax.broadcasted_iota(jnp.int32, (1, G), 1)
          ).astype(jnp.float32)
    hcat = jnp.concatenate([h1_ref[...], h2_ref[...], h3_ref[...]], axis=1)
    acc_ref[...] += jnp.dot(oh.T, hcat, preferred_element_type=jnp.float32)

    @pl.when(i == pl.num_programs(0) - 1)
    def _():
        z = jnp.dot(acc_ref[...], cw1_ref[...],
                    preferred_element_type=jnp.float32) + cb1_ref[...]
        z = z * _BN_SCALE * g_ref[...] + be_ref[...]
        z = jnp.maximum(z, 0.0)
        o_ref[...] = jnp.dot(z, cw2_ref[...],
                             preferred_element_type=jnp.float32) + cb2_ref[...]


_pool_cls = pl.pallas_call(
    _pool_cls_body,
    grid=(NBLK,),
    in_specs=[
        pl.BlockSpec((BR, 1), lambda i: (i, 0)),
        pl.BlockSpec((BR, H), lambda i: (i, 0)),
        pl.BlockSpec((BR, H), lambda i: (i, 0)),
        pl.BlockSpec((BR, H), lambda i: (i, 0)),
        pl.BlockSpec((3 * H, 2 * H), lambda i: (0, 0)),
        pl.BlockSpec((1, 2 * H), lambda i: (0, 0)),
        pl.BlockSpec((1, 2 * H), lambda i: (0, 0)),
        pl.BlockSpec((1, 2 * H), lambda i: (0, 0)),
        pl.BlockSpec((2 * H, 128), lambda i: (0, 0)),
        pl.BlockSpec((1, 128), lambda i: (0, 0)),
    ],
    out_specs=pl.BlockSpec((G, 128), lambda i: (0, 0)),
    out_shape=jax.ShapeDtypeStruct((G, 128), jnp.float32),
    scratch_shapes=[pltpu.VMEM((G, 3 * H), jnp.float32)],
)


def kernel(x, edge_index, batch, W1_0, b1_0, W2_0, b2_0, W1_1, b1_1, W2_1,
           b2_1, W1_2, b1_2, W2_2, b2_2, cW1, cb1, bn_gamma, bn_beta, cW2,
           cb2):
    src = edge_index[0]
    dst = edge_index[1]
    params = [(W1_0, b1_0, W2_0, b2_0), (W1_1, b1_1, W2_1, b2_1),
              (W1_2, b1_2, W2_2, b2_2)]

    h = x
    hs = []
    for (W1, b1, W2, b2) in params:
        agg = _sc_agg(h, src, dst)
        h = _mlp(h, agg, agg, W1, b1.reshape(1, H), W2, b2.reshape(1, H))
        hs.append(h)

    cW2p = jnp.zeros((2 * H, 128), jnp.float32).at[:, :NC].set(cW2)
    cb2p = jnp.zeros((1, 128), jnp.float32).at[0, :NC].set(cb2)
    out = _pool_cls(batch.reshape(N, 1), hs[0], hs[1], hs[2], cW1,
                    cb1.reshape(1, 2 * H), bn_gamma.reshape(1, 2 * H),
                    bn_beta.reshape(1, 2 * H), cW2p, cb2p)
    return out[:, :NC]


# speedup vs baseline: 2.8562x; 1.5553x over previous
import functools

import jax
import jax.numpy as jnp
import numpy as np
from jax import lax
from jax.experimental import pallas as pl
from jax.experimental.pallas import tpu as pltpu
from jax.experimental.pallas import tpu_sc as plsc

N = 10000
E = 320000
D = 128
H = 128
G = 64
NC = 2

NCORES = 2
NSUB = 16
NW = NCORES * NSUB
EPW = E // NW
CH = 128
NFULL = EPW // CH
TAIL = EPW - NFULL * CH
RPT = 640

_sc_mesh = plsc.VectorSubcoreMesh(core_axis_name="c", subcore_axis_name="s")


@functools.partial(
    pl.kernel,
    out_type=jax.ShapeDtypeStruct((2 * N, H), jnp.float32),
    mesh=_sc_mesh,
    scratch_types=[
        pltpu.VMEM((CH,), jnp.int32),
        pltpu.VMEM((CH,), jnp.int32),
        pltpu.VMEM((CH,), jnp.int32),
        pltpu.VMEM((CH,), jnp.int32),
        pltpu.VMEM((CH, H), jnp.float32),
        pltpu.VMEM((CH, H), jnp.float32),
        pltpu.VMEM((TAIL,), jnp.int32),
        pltpu.VMEM((TAIL,), jnp.int32),
        pltpu.VMEM((TAIL, H), jnp.float32),
        pltpu.SemaphoreType.DMA,
        pltpu.SemaphoreType.DMA,
        pltpu.SemaphoreType.DMA,
        pltpu.VMEM_SHARED((N, H), jnp.float32),
    ],
)
def _sc_agg(h_hbm, src_hbm, dst_hbm, out_hbm,
            sidx0, didx0, sidx1, didx1, rows0, rows1,
            sidx2, didx2, rows2, gsem0, gsem1, tsem, agg_sh):
    cid = lax.axis_index("c")
    sid = lax.axis_index("s")
    wid = cid * NSUB + sid
    base = wid * EPW

    ZB = 128
    def _zrow(r, carry):
        for c8 in range(H // 16):
            rows0[r, pl.ds(c8 * 16, 16)] = jnp.zeros((16,), jnp.float32)
        return carry
    lax.fori_loop(0, ZB, _zrow, 0)
    row0 = sid * RPT

    @pl.when(sid < NSUB - 1)
    def _():
        for t in range(RPT // ZB):
            pltpu.sync_copy(rows0.at[pl.ds(0, ZB)],
                            agg_sh.at[pl.ds(row0 + t * ZB, ZB)])

    @pl.when(sid == NSUB - 1)
    def _():
        for t in range(3):
            pltpu.sync_copy(rows0.at[pl.ds(0, ZB)],
                            agg_sh.at[pl.ds(row0 + t * ZB, ZB)])
        last = N - (NSUB - 1) * RPT - 3 * ZB
        pltpu.sync_copy(rows0.at[pl.ds(0, last)],
                        agg_sh.at[pl.ds(row0 + 3 * ZB, last)])
    plsc.subcore_barrier()

    def _idx(j, sb, db):
        off = base + j * CH
        pltpu.sync_copy(src_hbm.at[pl.ds(off, CH)], sb)
        pltpu.sync_copy(dst_hbm.at[pl.ds(off, CH)], db)

    _idx(0, sidx0, didx0)
    pltpu.async_copy(h_hbm.at[sidx0], rows0, gsem0)

    def _pair(t, carry):
        j0 = 2 * t
        _idx(j0 + 1, sidx1, didx1)
        pltpu.async_copy(h_hbm.at[sidx1], rows1, gsem1)
        pltpu.make_async_copy(h_hbm.at[sidx0], rows0, gsem0).wait()
        pltpu.sync_copy(rows0, agg_sh.at[didx0], add=True)
        _idx(j0 + 2, sidx0, didx0)
        pltpu.async_copy(h_hbm.at[sidx0], rows0, gsem0)
        pltpu.make_async_copy(h_hbm.at[sidx1], rows1, gsem1).wait()
        pltpu.sync_copy(rows1, agg_sh.at[didx1], add=True)
        return carry
    lax.fori_loop(0, NFULL // 2 - 1, _pair, 0)

    _idx(NFULL - 1, sidx1, didx1)
    pltpu.async_copy(h_hbm.at[sidx1], rows1, gsem1)
    pltpu.make_async_copy(h_hbm.at[sidx0], rows0, gsem0).wait()
    pltpu.sync_copy(rows0, agg_sh.at[didx0], add=True)
    off = base + NFULL * CH
    pltpu.sync_copy(src_hbm.at[pl.ds(off, TAIL)], sidx2)
    pltpu.sync_copy(dst_hbm.at[pl.ds(off, TAIL)], didx2)
    pltpu.async_copy(h_hbm.at[sidx2], rows2, tsem)
    pltpu.make_async_copy(h_hbm.at[sidx1], rows1, gsem1).wait()
    pltpu.sync_copy(rows1, agg_sh.at[didx1], add=True)
    pltpu.make_async_copy(h_hbm.at[sidx2], rows2, tsem).wait()
    pltpu.sync_copy(rows2, agg_sh.at[didx2], add=True)

    plsc.subcore_barrier()

    @pl.when(sid < NSUB - 1)
    def _():
        pltpu.sync_copy(agg_sh.at[pl.ds(row0, RPT)],
                        out_hbm.at[pl.ds(cid * N + row0, RPT)])

    @pl.when(sid == NSUB - 1)
    def _():
        pltpu.sync_copy(agg_sh.at[pl.ds(row0, N - (NSUB - 1) * RPT)],
                        out_hbm.at[pl.ds(cid * N + row0, N - (NSUB - 1) * RPT)])


BR = 1000
NBLK = N // BR


def _mlp_body(h_ref, a0_ref, a1_ref, w1_ref, b1_ref, w2_ref, b2_ref, o_ref):
    z = h_ref[...] + a0_ref[...] + a1_ref[...]
    t = jnp.maximum(
        jnp.dot(z, w1_ref[...], preferred_element_type=jnp.float32) + b1_ref[...],
        0.0)
    o_ref[...] = jnp.maximum(
        jnp.dot(t, w2_ref[...], preferred_element_type=jnp.float32) + b2_ref[...],
        0.0)


_mlp = pl.pallas_call(
    _mlp_body,
    grid=(NBLK,),
    in_specs=[
        pl.BlockSpec((BR, H), lambda i: (i, 0)),
        pl.BlockSpec((BR, H), lambda i: (i, 0)),
        pl.BlockSpec((BR, H), lambda i: (NBLK + i, 0)),
        pl.BlockSpec((H, H), lambda i: (0, 0)),
        pl.BlockSpec((1, H), lambda i: (0, 0)),
        pl.BlockSpec((H, H), lambda i: (0, 0)),
        pl.BlockSpec((1, H), lambda i: (0, 0)),
    ],
    out_specs=pl.BlockSpec((BR, H), lambda i: (i, 0)),
    out_shape=jax.ShapeDtypeStruct((N, H), jnp.float32),
)

_BN_SCALE = float(1.0 / np.sqrt(1.0 + 1e-5))


def _pool_cls_body(b_ref, h1_ref, h2_ref, h3_ref, cw1_ref, cb1_ref,
                   g_ref, be_ref, cw2_ref, cb2_ref, o_ref, acc_ref):
    i = pl.program_id(0)

    @pl.when(i == 0)
    def _():
        acc_ref[...] = jnp.zeros_like(acc_ref)

    oh = (b_ref[...] == lax.broadcasted_iota(jnp.int32, (1, G), 1)
          ).astype(jnp.float32)
    hcat = jnp.concatenate([h1_ref[...], h2_ref[...], h3_ref[...]], axis=1)
    acc_ref[...] += jnp.dot(oh.T, hcat, preferred_element_type=jnp.float32)

    @pl.when(i == pl.num_programs(0) - 1)
    def _():
        z = jnp.dot(acc_ref[...], cw1_ref[...],
                    preferred_element_type=jnp.float32) + cb1_ref[...]
        z = z * _BN_SCALE * g_ref[...] + be_ref[...]
        z = jnp.maximum(z, 0.0)
        o_ref[...] = jnp.dot(z, cw2_ref[...],
                             preferred_element_type=jnp.float32) + cb2_ref[...]


_pool_cls = pl.pallas_call(
    _pool_cls_body,
    grid=(NBLK,),
    in_specs=[
        pl.BlockSpec((BR, 1), lambda i: (i, 0)),
        pl.BlockSpec((BR, H), lambda i: (i, 0)),
        pl.BlockSpec((BR, H), lambda i: (i, 0)),
        pl.BlockSpec((BR, H), lambda i: (i, 0)),
        pl.BlockSpec((3 * H, 2 * H), lambda i: (0, 0)),
        pl.BlockSpec((1, 2 * H), lambda i: (0, 0)),
        pl.BlockSpec((1, 2 * H), lambda i: (0, 0)),
        pl.BlockSpec((1, 2 * H), lambda i: (0, 0)),
        pl.BlockSpec((2 * H, 128), lambda i: (0, 0)),
        pl.BlockSpec((1, 128), lambda i: (0, 0)),
    ],
    out_specs=pl.BlockSpec((G, 128), lambda i: (0, 0)),
    out_shape=jax.ShapeDtypeStruct((G, 128), jnp.float32),
    scratch_shapes=[pltpu.VMEM((G, 3 * H), jnp.float32)],
)


def kernel(x, edge_index, batch, W1_0, b1_0, W2_0, b2_0, W1_1, b1_1, W2_1,
           b2_1, W1_2, b1_2, W2_2, b2_2, cW1, cb1, bn_gamma, bn_beta, cW2,
           cb2):
    src = edge_index[0]
    dst = edge_index[1]
    params = [(W1_0, b1_0, W2_0, b2_0), (W1_1, b1_1, W2_1, b2_1),
              (W1_2, b1_2, W2_2, b2_2)]

    h = x
    hs = []
    for (W1, b1, W2, b2) in params:
        agg = _sc_agg(h, src, dst)
        h = _mlp(h, agg, agg, W1, b1.reshape(1, H), W2, b2.reshape(1, H))
        hs.append(h)

    cW2p = jnp.zeros((2 * H, 128), jnp.float32).at[:, :NC].set(cW2)
    cb2p = jnp.zeros((1, 128), jnp.float32).at[0, :NC].set(cb2)
    out = _pool_cls(batch.reshape(N, 1), hs[0], hs[1], hs[2], cW1,
                    cb1.reshape(1, 2 * H), bn_gamma.reshape(1, 2 * H),
                    bn_beta.reshape(1, 2 * H), cW2p, cb2p)
    return out[:, :NC]


# speedup vs baseline: 2.8602x; 1.0014x over previous
import functools

import jax
import jax.numpy as jnp
import numpy as np
from jax import lax
from jax.experimental import pallas as pl
from jax.experimental.pallas import tpu as pltpu
from jax.experimental.pallas import tpu_sc as plsc

N = 10000
E = 320000
D = 128
H = 128
G = 64
NC = 2

NCORES = 2
NSUB = 16
NW = NCORES * NSUB
EPW = E // NW
CH = 128
NFULL = EPW // CH
TAIL = EPW - NFULL * CH
RPT = 640

_sc_mesh = plsc.VectorSubcoreMesh(core_axis_name="c", subcore_axis_name="s")


@functools.partial(
    pl.kernel,
    out_type=jax.ShapeDtypeStruct((2 * N, H), jnp.float32),
    mesh=_sc_mesh,
    scratch_types=[
        [pltpu.VMEM((CH,), jnp.int32) for _ in range(3)],
        [pltpu.VMEM((CH,), jnp.int32) for _ in range(3)],
        [pltpu.VMEM((CH, H), jnp.float32) for _ in range(3)],
        pltpu.VMEM((TAIL,), jnp.int32),
        pltpu.VMEM((TAIL,), jnp.int32),
        [pltpu.SemaphoreType.DMA for _ in range(3)],
        pltpu.SemaphoreType.DMA,
        pltpu.VMEM_SHARED((N, H), jnp.float32),
    ],
)
def _sc_agg(h_hbm, src_hbm, dst_hbm, out_hbm,
            sidxs, didxs, rows, sidx2, didx2, gsems, tsem, agg_sh):
    rows0 = rows[0]
    cid = lax.axis_index("c")
    sid = lax.axis_index("s")
    wid = cid * NSUB + sid
    base = wid * EPW

    ZB = 128
    def _zrow(r, carry):
        for c8 in range(H // 16):
            rows0[r, pl.ds(c8 * 16, 16)] = jnp.zeros((16,), jnp.float32)
        return carry
    lax.fori_loop(0, ZB, _zrow, 0)
    row0 = sid * RPT

    @pl.when(sid < NSUB - 1)
    def _():
        for t in range(RPT // ZB):
            pltpu.sync_copy(rows0.at[pl.ds(0, ZB)],
                            agg_sh.at[pl.ds(row0 + t * ZB, ZB)])

    @pl.when(sid == NSUB - 1)
    def _():
        for t in range(3):
            pltpu.sync_copy(rows0.at[pl.ds(0, ZB)],
                            agg_sh.at[pl.ds(row0 + t * ZB, ZB)])
        last = N - (NSUB - 1) * RPT - 3 * ZB
        pltpu.sync_copy(rows0.at[pl.ds(0, last)],
                        agg_sh.at[pl.ds(row0 + 3 * ZB, last)])
    plsc.subcore_barrier()

    def _idx(j, k):
        off = base + j * CH
        pltpu.sync_copy(src_hbm.at[pl.ds(off, CH)], sidxs[k])
        pltpu.sync_copy(dst_hbm.at[pl.ds(off, CH)], didxs[k])

    def _gather(k):
        pltpu.async_copy(h_hbm.at[sidxs[k]], rows[k], gsems[k])

    def _gwait(k):
        pltpu.make_async_copy(h_hbm.at[sidxs[k]], rows[k], gsems[k]).wait()

    for k in range(2):
        _idx(k, k)
        _gather(k)

    def _trip(t, carry):
        j0 = 3 * t
        for k in range(3):
            kf = (k + 2) % 3
            _idx(j0 + k + 2, kf)
            _gather(kf)
            _gwait(k)
            pltpu.sync_copy(rows[k], agg_sh.at[didxs[k]], add=True)
        return carry
    lax.fori_loop(0, NFULL // 3 - 1, _trip, 0)

    _idx(NFULL - 1, 2)
    _gather(2)
    _gwait(0)
    pltpu.sync_copy(rows[0], agg_sh.at[didxs[0]], add=True)
    off = base + NFULL * CH
    pltpu.sync_copy(src_hbm.at[pl.ds(off, TAIL)], sidx2)
    pltpu.sync_copy(dst_hbm.at[pl.ds(off, TAIL)], didx2)
    tail_dst = rows[0].at[pl.ds(0, TAIL)]
    pltpu.async_copy(h_hbm.at[sidx2], tail_dst, tsem)
    _gwait(1)
    pltpu.sync_copy(rows[1], agg_sh.at[didxs[1]], add=True)
    _gwait(2)
    pltpu.sync_copy(rows[2], agg_sh.at[didxs[2]], add=True)
    pltpu.make_async_copy(h_hbm.at[sidx2], tail_dst, tsem).wait()
    pltpu.sync_copy(tail_dst, agg_sh.at[didx2], add=True)

    plsc.subcore_barrier()

    @pl.when(sid < NSUB - 1)
    def _():
        pltpu.sync_copy(agg_sh.at[pl.ds(row0, RPT)],
                        out_hbm.at[pl.ds(cid * N + row0, RPT)])

    @pl.when(sid == NSUB - 1)
    def _():
        pltpu.sync_copy(agg_sh.at[pl.ds(row0, N - (NSUB - 1) * RPT)],
                        out_hbm.at[pl.ds(cid * N + row0, N - (NSUB - 1) * RPT)])


BR = 1000
NBLK = N // BR


def _mlp_body(h_ref, a0_ref, a1_ref, w1_ref, b1_ref, w2_ref, b2_ref, o_ref):
    z = h_ref[...] + a0_ref[...] + a1_ref[...]
    t = jnp.maximum(
        jnp.dot(z, w1_ref[...], preferred_element_type=jnp.float32) + b1_ref[...],
        0.0)
    o_ref[...] = jnp.maximum(
        jnp.dot(t, w2_ref[...], preferred_element_type=jnp.float32) + b2_ref[...],
        0.0)


_mlp = pl.pallas_call(
    _mlp_body,
    grid=(NBLK,),
    in_specs=[
        pl.BlockSpec((BR, H), lambda i: (i, 0)),
        pl.BlockSpec((BR, H), lambda i: (i, 0)),
        pl.BlockSpec((BR, H), lambda i: (NBLK + i, 0)),
        pl.BlockSpec((H, H), lambda i: (0, 0)),
        pl.BlockSpec((1, H), lambda i: (0, 0)),
        pl.BlockSpec((H, H), lambda i: (0, 0)),
        pl.BlockSpec((1, H), lambda i: (0, 0)),
    ],
    out_specs=pl.BlockSpec((BR, H), lambda i: (i, 0)),
    out_shape=jax.ShapeDtypeStruct((N, H), jnp.float32),
)

_BN_SCALE = float(1.0 / np.sqrt(1.0 + 1e-5))


def _pool_cls_body(b_ref, h1_ref, h2_ref, h3_ref, cw1_ref, cb1_ref,
                   g_ref, be_ref, cw2_ref, cb2_ref, o_ref, acc_ref):
    i = pl.program_id(0)

    @pl.when(i == 0)
    def _():
        acc_ref[...] = jnp.zeros_like(acc_ref)

    oh = (b_ref[...] == lax.broadcasted_iota(jnp.int32, (1, G), 1)
          ).astype(jnp.float32)
    hcat = jnp.concatenate([h1_ref[...], h2_ref[...], h3_ref[...]], axis=1)
    acc_ref[...] += jnp.dot(oh.T, hcat, preferred_element_type=jnp.float32)

    @pl.when(i == pl.num_programs(0) - 1)
    def _():
        z = jnp.dot(acc_ref[...], cw1_ref[...],
                    preferred_element_type=jnp.float32) + cb1_ref[...]
        z = z * _BN_SCALE * g_ref[...] + be_ref[...]
        z = jnp.maximum(z, 0.0)
        o_ref[...] = jnp.dot(z, cw2_ref[...],
                             preferred_element_type=jnp.float32) + cb2_ref[...]


_pool_cls = pl.pallas_call(
    _pool_cls_body,
    grid=(NBLK,),
    in_specs=[
        pl.BlockSpec((BR, 1), lambda i: (i, 0)),
        pl.BlockSpec((BR, H), lambda i: (i, 0)),
        pl.BlockSpec((BR, H), lambda i: (i, 0)),
        pl.BlockSpec((BR, H), lambda i: (i, 0)),
        pl.BlockSpec((3 * H, 2 * H), lambda i: (0, 0)),
        pl.BlockSpec((1, 2 * H), lambda i: (0, 0)),
        pl.BlockSpec((1, 2 * H), lambda i: (0, 0)),
        pl.BlockSpec((1, 2 * H), lambda i: (0, 0)),
        pl.BlockSpec((2 * H, 128), lambda i: (0, 0)),
        pl.BlockSpec((1, 128), lambda i: (0, 0)),
    ],
    out_specs=pl.BlockSpec((G, 128), lambda i: (0, 0)),
    out_shape=jax.ShapeDtypeStruct((G, 128), jnp.float32),
    scratch_shapes=[pltpu.VMEM((G, 3 * H), jnp.float32)],
)


def kernel(x, edge_index, batch, W1_0, b1_0, W2_0, b2_0, W1_1, b1_1, W2_1,
           b2_1, W1_2, b1_2, W2_2, b2_2, cW1, cb1, bn_gamma, bn_beta, cW2,
           cb2):
    src = edge_index[0]
    dst = edge_index[1]
    params = [(W1_0, b1_0, W2_0, b2_0), (W1_1, b1_1, W2_1, b2_1),
              (W1_2, b1_2, W2_2, b2_2)]

    h = x
    hs = []
    for (W1, b1, W2, b2) in params:
        agg = _sc_agg(h, src, dst)
        h = _mlp(h, agg, agg, W1, b1.reshape(1, H), W2, b2.reshape(1, H))
        hs.append(h)

    cW2p = jnp.zeros((2 * H, 128), jnp.float32).at[:, :NC].set(cW2)
    cb2p = jnp.zeros((1, 128), jnp.float32).at[0, :NC].set(cb2)
    out = _pool_cls(batch.reshape(N, 1), hs[0], hs[1], hs[2], cW1,
                    cb1.reshape(1, 2 * H), bn_gamma.reshape(1, 2 * H),
                    bn_beta.reshape(1, 2 * H), cW2p, cb2p)
    return out[:, :NC]


# speedup vs baseline: 3.4690x; 1.2128x over previous
import functools

import jax
import jax.numpy as jnp
import numpy as np
from jax import lax
from jax.experimental import pallas as pl
from jax.experimental.pallas import tpu as pltpu
from jax.experimental.pallas import tpu_sc as plsc

N = 10000
E = 320000
D = 128
H = 128
G = 64
NC = 2

NCORES = 2
NSUB = 16
NW = NCORES * NSUB
EPW = E // NW
CH = 128
NFULL = EPW // CH
TAIL = EPW - NFULL * CH
RPT = 640

_sc_mesh = plsc.VectorSubcoreMesh(core_axis_name="c", subcore_axis_name="s")


@functools.partial(
    pl.kernel,
    out_type=jax.ShapeDtypeStruct((2 * N, H), jnp.float32),
    mesh=_sc_mesh,
    scratch_types=[
        [pltpu.VMEM((CH,), jnp.int32) for _ in range(3)],
        [pltpu.VMEM((CH,), jnp.int32) for _ in range(3)],
        [pltpu.VMEM((CH, H), jnp.float32) for _ in range(3)],
        pltpu.VMEM((TAIL,), jnp.int32),
        pltpu.VMEM((TAIL,), jnp.int32),
        [pltpu.SemaphoreType.DMA for _ in range(3)],
        [pltpu.SemaphoreType.DMA for _ in range(3)],
        pltpu.SemaphoreType.DMA,
        pltpu.VMEM_SHARED((N, H), jnp.float32),
    ],
)
def _sc_agg(h_hbm, src_hbm, dst_hbm, out_hbm,
            sidxs, didxs, rows, sidx2, didx2, gsems, isems, tsem, agg_sh):
    rows0 = rows[0]
    cid = lax.axis_index("c")
    sid = lax.axis_index("s")
    wid = cid * NSUB + sid
    base = wid * EPW

    ZB = 128
    def _zrow(r, carry):
        for c8 in range(H // 16):
            rows0[r, pl.ds(c8 * 16, 16)] = jnp.zeros((16,), jnp.float32)
        return carry
    lax.fori_loop(0, ZB, _zrow, 0)
    row0 = sid * RPT

    @pl.when(sid < NSUB - 1)
    def _():
        for t in range(RPT // ZB):
            pltpu.sync_copy(rows0.at[pl.ds(0, ZB)],
                            agg_sh.at[pl.ds(row0 + t * ZB, ZB)])

    @pl.when(sid == NSUB - 1)
    def _():
        for t in range(3):
            pltpu.sync_copy(rows0.at[pl.ds(0, ZB)],
                            agg_sh.at[pl.ds(row0 + t * ZB, ZB)])
        last = N - (NSUB - 1) * RPT - 3 * ZB
        pltpu.sync_copy(rows0.at[pl.ds(0, last)],
                        agg_sh.at[pl.ds(row0 + 3 * ZB, last)])
    plsc.subcore_barrier()

    def _idx(j, k):
        off = base + j * CH
        pltpu.async_copy(src_hbm.at[pl.ds(off, CH)], sidxs[k], isems[k])
        pltpu.async_copy(dst_hbm.at[pl.ds(off, CH)], didxs[k], isems[k])

    def _iwait(j, k):
        off = base + j * CH
        pltpu.make_async_copy(src_hbm.at[pl.ds(off, CH)], sidxs[k],
                              isems[k]).wait()
        pltpu.make_async_copy(dst_hbm.at[pl.ds(off, CH)], didxs[k],
                              isems[k]).wait()

    def _gather(k):
        pltpu.async_copy(h_hbm.at[sidxs[k]], rows[k], gsems[k])

    def _gwait(k):
        pltpu.make_async_copy(h_hbm.at[sidxs[k]], rows[k], gsems[k]).wait()

    for k in range(3):
        _idx(k, k)
    for k in range(2):
        _iwait(k, k)
        _gather(k)

    def _trip(t, carry):
        j0 = 3 * t
        for k in range(3):
            kf = (k + 2) % 3
            _iwait(j0 + k + 2, kf)
            _gather(kf)
            _gwait(k)
            pltpu.sync_copy(rows[k], agg_sh.at[didxs[k]], add=True)
            _idx(j0 + k + 3, k)
        return carry
    lax.fori_loop(0, NFULL // 3 - 1, _trip, 0)

    _iwait(NFULL - 1, 2)
    _gather(2)
    _gwait(0)
    pltpu.sync_copy(rows[0], agg_sh.at[didxs[0]], add=True)
    off = base + NFULL * CH
    pltpu.sync_copy(src_hbm.at[pl.ds(off, TAIL)], sidx2)
    pltpu.sync_copy(dst_hbm.at[pl.ds(off, TAIL)], didx2)
    tail_dst = rows[0].at[pl.ds(0, TAIL)]
    pltpu.async_copy(h_hbm.at[sidx2], tail_dst, tsem)
    _gwait(1)
    pltpu.sync_copy(rows[1], agg_sh.at[didxs[1]], add=True)
    _gwait(2)
    pltpu.sync_copy(rows[2], agg_sh.at[didxs[2]], add=True)
    pltpu.make_async_copy(h_hbm.at[sidx2], tail_dst, tsem).wait()
    pltpu.sync_copy(tail_dst, agg_sh.at[didx2], add=True)

    plsc.subcore_barrier()

    @pl.when(sid < NSUB - 1)
    def _():
        pltpu.sync_copy(agg_sh.at[pl.ds(row0, RPT)],
                        out_hbm.at[pl.ds(cid * N + row0, RPT)])

    @pl.when(sid == NSUB - 1)
    def _():
        pltpu.sync_copy(agg_sh.at[pl.ds(row0, N - (NSUB - 1) * RPT)],
                        out_hbm.at[pl.ds(cid * N + row0, N - (NSUB - 1) * RPT)])


BR = 1000
NBLK = N // BR


def _mlp_body(h_ref, a0_ref, a1_ref, w1_ref, b1_ref, w2_ref, b2_ref, o_ref):
    z = h_ref[...] + a0_ref[...] + a1_ref[...]
    t = jnp.maximum(
        jnp.dot(z, w1_ref[...], preferred_element_type=jnp.float32) + b1_ref[...],
        0.0)
    o_ref[...] = jnp.maximum(
        jnp.dot(t, w2_ref[...], preferred_element_type=jnp.float32) + b2_ref[...],
        0.0)


_mlp = pl.pallas_call(
    _mlp_body,
    grid=(NBLK,),
    in_specs=[
        pl.BlockSpec((BR, H), lambda i: (i, 0)),
        pl.BlockSpec((BR, H), lambda i: (i, 0)),
        pl.BlockSpec((BR, H), lambda i: (NBLK + i, 0)),
        pl.BlockSpec((H, H), lambda i: (0, 0)),
        pl.BlockSpec((1, H), lambda i: (0, 0)),
        pl.BlockSpec((H, H), lambda i: (0, 0)),
        pl.BlockSpec((1, H), lambda i: (0, 0)),
    ],
    out_specs=pl.BlockSpec((BR, H), lambda i: (i, 0)),
    out_shape=jax.ShapeDtypeStruct((N, H), jnp.float32),
)

_BN_SCALE = float(1.0 / np.sqrt(1.0 + 1e-5))


def _pool_cls_body(b_ref, h1_ref, h2_ref, h3_ref, cw1_ref, cb1_ref,
                   g_ref, be_ref, cw2_ref, cb2_ref, o_ref, acc_ref):
    i = pl.program_id(0)

    @pl.when(i == 0)
    def _():
        acc_ref[...] = jnp.zeros_like(acc_ref)

    oh = (b_ref[...] == lax.broadcasted_iota(jnp.int32, (1, G), 1)
          ).astype(jnp.float32)
    hcat = jnp.concatenate([h1_ref[...], h2_ref[...], h3_ref[...]], axis=1)
    acc_ref[...] += jnp.dot(oh.T, hcat, preferred_element_type=jnp.float32)

    @pl.when(i == pl.num_programs(0) - 1)
    def _():
        z = jnp.dot(acc_ref[...], cw1_ref[...],
                    preferred_element_type=jnp.float32) + cb1_ref[...]
        z = z * _BN_SCALE * g_ref[...] + be_ref[...]
        z = jnp.maximum(z, 0.0)
        o_ref[...] = jnp.dot(z, cw2_ref[...],
                             preferred_element_type=jnp.float32) + cb2_ref[...]


_pool_cls = pl.pallas_call(
    _pool_cls_body,
    grid=(NBLK,),
    in_specs=[
        pl.BlockSpec((BR, 1), lambda i: (i, 0)),
        pl.BlockSpec((BR, H), lambda i: (i, 0)),
        pl.BlockSpec((BR, H), lambda i: (i, 0)),
        pl.BlockSpec((BR, H), lambda i: (i, 0)),
        pl.BlockSpec((3 * H, 2 * H), lambda i: (0, 0)),
        pl.BlockSpec((1, 2 * H), lambda i: (0, 0)),
        pl.BlockSpec((1, 2 * H), lambda i: (0, 0)),
        pl.BlockSpec((1, 2 * H), lambda i: (0, 0)),
        pl.BlockSpec((2 * H, 128), lambda i: (0, 0)),
        pl.BlockSpec((1, 128), lambda i: (0, 0)),
    ],
    out_specs=pl.BlockSpec((G, 128), lambda i: (0, 0)),
    out_shape=jax.ShapeDtypeStruct((G, 128), jnp.float32),
    scratch_shapes=[pltpu.VMEM((G, 3 * H), jnp.float32)],
)


def kernel(x, edge_index, batch, W1_0, b1_0, W2_0, b2_0, W1_1, b1_1, W2_1,
           b2_1, W1_2, b1_2, W2_2, b2_2, cW1, cb1, bn_gamma, bn_beta, cW2,
           cb2):
    src = edge_index[0]
    dst = edge_index[1]
    params = [(W1_0, b1_0, W2_0, b2_0), (W1_1, b1_1, W2_1, b2_1),
              (W1_2, b1_2, W2_2, b2_2)]

    h = x
    hs = []
    for (W1, b1, W2, b2) in params:
        agg = _sc_agg(h, src, dst)
        h = _mlp(h, agg, agg, W1, b1.reshape(1, H), W2, b2.reshape(1, H))
        hs.append(h)

    cW2p = jnp.zeros((2 * H, 128), jnp.float32).at[:, :NC].set(cW2)
    cb2p = jnp.zeros((1, 128), jnp.float32).at[0, :NC].set(cb2)
    out = _pool_cls(batch.reshape(N, 1), hs[0], hs[1], hs[2], cW1,
                    cb1.reshape(1, 2 * H), bn_gamma.reshape(1, 2 * H),
                    bn_beta.reshape(1, 2 * H), cW2p, cb2p)
    return out[:, :NC]


# speedup vs baseline: 3.8972x; 1.1234x over previous
import functools

import jax
import jax.numpy as jnp
import numpy as np
from jax import lax
from jax.experimental import pallas as pl
from jax.experimental.pallas import tpu as pltpu
from jax.experimental.pallas import tpu_sc as plsc

N = 10000
E = 320000
D = 128
H = 128
G = 64
NC = 2

NCORES = 2
NSUB = 16
NW = NCORES * NSUB
EPW = E // NW
CH = 128
NFULL = EPW // CH
TAIL = EPW - NFULL * CH
RPT = 640

_sc_mesh = plsc.VectorSubcoreMesh(core_axis_name="c", subcore_axis_name="s")


@functools.partial(
    pl.kernel,
    out_type=jax.ShapeDtypeStruct((2 * N, H), jnp.float32),
    mesh=_sc_mesh,
    scratch_types=[
        [pltpu.VMEM((CH,), jnp.int32) for _ in range(3)],
        [pltpu.VMEM((CH,), jnp.int32) for _ in range(6)],
        [pltpu.VMEM((CH, H), jnp.float32) for _ in range(3)],
        pltpu.VMEM((TAIL,), jnp.int32),
        pltpu.VMEM((TAIL,), jnp.int32),
        [pltpu.SemaphoreType.DMA for _ in range(3)],
        [pltpu.SemaphoreType.DMA for _ in range(3)],
        [pltpu.SemaphoreType.DMA for _ in range(2)],
        pltpu.SemaphoreType.DMA,
        pltpu.VMEM_SHARED((N, H), jnp.float32),
    ],
)
def _sc_agg(h_hbm, src_hbm, dst_hbm, out_hbm,
            sidxs, didxs, rows, sidx2, didx2, gsems, isems, ssems, tsem,
            agg_sh):
    rows0 = rows[0]
    cid = lax.axis_index("c")
    sid = lax.axis_index("s")
    wid = cid * NSUB + sid
    base = wid * EPW

    ZB = 128
    def _zrow(r, carry):
        for c8 in range(H // 16):
            rows0[r, pl.ds(c8 * 16, 16)] = jnp.zeros((16,), jnp.float32)
        return carry
    lax.fori_loop(0, ZB, _zrow, 0)
    row0 = sid * RPT

    @pl.when(sid < NSUB - 1)
    def _():
        for t in range(RPT // ZB):
            pltpu.sync_copy(rows0.at[pl.ds(0, ZB)],
                            agg_sh.at[pl.ds(row0 + t * ZB, ZB)])

    @pl.when(sid == NSUB - 1)
    def _():
        for t in range(3):
            pltpu.sync_copy(rows0.at[pl.ds(0, ZB)],
                            agg_sh.at[pl.ds(row0 + t * ZB, ZB)])
        last = N - (NSUB - 1) * RPT - 3 * ZB
        pltpu.sync_copy(rows0.at[pl.ds(0, last)],
                        agg_sh.at[pl.ds(row0 + 3 * ZB, last)])
    plsc.subcore_barrier()

    def _idx(j, r):
        off = base + j * CH
        pltpu.async_copy(src_hbm.at[pl.ds(off, CH)], sidxs[r % 3],
                         isems[r % 3])
        pltpu.async_copy(dst_hbm.at[pl.ds(off, CH)], didxs[r % 6],
                         isems[r % 3])

    def _iwait(j, r):
        off = base + j * CH
        pltpu.make_async_copy(src_hbm.at[pl.ds(off, CH)], sidxs[r % 3],
                              isems[r % 3]).wait()
        pltpu.make_async_copy(dst_hbm.at[pl.ds(off, CH)], didxs[r % 6],
                              isems[r % 3]).wait()

    def _gather(r):
        pltpu.async_copy(h_hbm.at[sidxs[r % 3]], rows[r % 3], gsems[r % 3])

    def _gwait(r):
        pltpu.make_async_copy(h_hbm.at[sidxs[r % 3]], rows[r % 3],
                              gsems[r % 3]).wait()

    def _sstart(r):
        pltpu.async_copy(rows[r % 3], agg_sh.at[didxs[r % 6]], ssems[r % 2],
                         add=True)

    def _swait(r):
        pltpu.make_async_copy(rows[r % 3], agg_sh.at[didxs[r % 6]],
                              ssems[r % 2]).wait()

    for j in range(3):
        _idx(j, j)
    for j in range(2):
        _iwait(j, j)
        _gather(j)

    def _slot(j, r):
        _gwait(r)
        _sstart(r)
        _idx(j + 3, r + 3)
        _swait(r + 5)
        _iwait(j + 2, r + 2)
        _gather(r + 2)

    _gwait(0); _sstart(0); _idx(3, 3); _iwait(2, 2); _gather(2)
    _gwait(1); _sstart(1); _idx(4, 4); _swait(0); _iwait(3, 3); _gather(3)

    def _six(t, carry):
        j0 = 6 * t + 2
        for k in range(6):
            _slot(j0 + k, 2 + k)
        return carry
    lax.fori_loop(0, 12, _six, 0)

    _gwait(74); _sstart(74); _idx(77, 77); _swait(73); _iwait(76, 76); _gather(76)
    _gwait(75); _sstart(75); _swait(74); _iwait(77, 77); _gather(77)
    _gwait(76); _sstart(76); _swait(75)
    off = base + NFULL * CH
    pltpu.sync_copy(src_hbm.at[pl.ds(off, TAIL)], sidx2)
    pltpu.sync_copy(dst_hbm.at[pl.ds(off, TAIL)], didx2)
    tail_dst = rows[0].at[pl.ds(0, TAIL)]
    pltpu.async_copy(h_hbm.at[sidx2], tail_dst, tsem)
    _gwait(77); _sstart(77); _swait(76)
    pltpu.make_async_copy(h_hbm.at[sidx2], tail_dst, tsem).wait()
    _swait(77)
    pltpu.sync_copy(tail_dst, agg_sh.at[didx2], add=True)

    plsc.subcore_barrier()

    @pl.when(sid < NSUB - 1)
    def _():
        pltpu.sync_copy(agg_sh.at[pl.ds(row0, RPT)],
                        out_hbm.at[pl.ds(cid * N + row0, RPT)])

    @pl.when(sid == NSUB - 1)
    def _():
        pltpu.sync_copy(agg_sh.at[pl.ds(row0, N - (NSUB - 1) * RPT)],
                        out_hbm.at[pl.ds(cid * N + row0, N - (NSUB - 1) * RPT)])


BR = 1000
NBLK = N // BR


def _mlp_body(h_ref, a0_ref, a1_ref, w1_ref, b1_ref, w2_ref, b2_ref, o_ref):
    z = h_ref[...] + a0_ref[...] + a1_ref[...]
    t = jnp.maximum(
        jnp.dot(z, w1_ref[...], preferred_element_type=jnp.float32) + b1_ref[...],
        0.0)
    o_ref[...] = jnp.maximum(
        jnp.dot(t, w2_ref[...], preferred_element_type=jnp.float32) + b2_ref[...],
        0.0)


_mlp = pl.pallas_call(
    _mlp_body,
    grid=(NBLK,),
    in_specs=[
        pl.BlockSpec((BR, H), lambda i: (i, 0)),
        pl.BlockSpec((BR, H), lambda i: (i, 0)),
        pl.BlockSpec((BR, H), lambda i: (NBLK + i, 0)),
        pl.BlockSpec((H, H), lambda i: (0, 0)),
        pl.BlockSpec((1, H), lambda i: (0, 0)),
        pl.BlockSpec((H, H), lambda i: (0, 0)),
        pl.BlockSpec((1, H), lambda i: (0, 0)),
    ],
    out_specs=pl.BlockSpec((BR, H), lambda i: (i, 0)),
    out_shape=jax.ShapeDtypeStruct((N, H), jnp.float32),
)

_BN_SCALE = float(1.0 / np.sqrt(1.0 + 1e-5))


def _pool_cls_body(b_ref, h1_ref, h2_ref, h3_ref, cw1_ref, cb1_ref,
                   g_ref, be_ref, cw2_ref, cb2_ref, o_ref, acc_ref):
    i = pl.program_id(0)

    @pl.when(i == 0)
    def _():
        acc_ref[...] = jnp.zeros_like(acc_ref)

    oh = (b_ref[...] == lax.broadcasted_iota(jnp.int32, (1, G), 1)
          ).astype(jnp.float32)
    hcat = jnp.concatenate([h1_ref[...], h2_ref[...], h3_ref[...]], axis=1)
    acc_ref[...] += jnp.dot(oh.T, hcat, preferred_element_type=jnp.float32)

    @pl.when(i == pl.num_programs(0) - 1)
    def _():
        z = jnp.dot(acc_ref[...], cw1_ref[...],
                    preferred_element_type=jnp.float32) + cb1_ref[...]
        z = z * _BN_SCALE * g_ref[...] + be_ref[...]
        z = jnp.maximum(z, 0.0)
        o_ref[...] = jnp.dot(z, cw2_ref[...],
                             preferred_element_type=jnp.float32) + cb2_ref[...]


_pool_cls = pl.pallas_call(
    _pool_cls_body,
    grid=(NBLK,),
    in_specs=[
        pl.BlockSpec((BR, 1), lambda i: (i, 0)),
        pl.BlockSpec((BR, H), lambda i: (i, 0)),
        pl.BlockSpec((BR, H), lambda i: (i, 0)),
        pl.BlockSpec((BR, H), lambda i: (i, 0)),
        pl.BlockSpec((3 * H, 2 * H), lambda i: (0, 0)),
        pl.BlockSpec((1, 2 * H), lambda i: (0, 0)),
        pl.BlockSpec((1, 2 * H), lambda i: (0, 0)),
        pl.BlockSpec((1, 2 * H), lambda i: (0, 0)),
        pl.BlockSpec((2 * H, 128), lambda i: (0, 0)),
        pl.BlockSpec((1, 128), lambda i: (0, 0)),
    ],
    out_specs=pl.BlockSpec((G, 128), lambda i: (0, 0)),
    out_shape=jax.ShapeDtypeStruct((G, 128), jnp.float32),
    scratch_shapes=[pltpu.VMEM((G, 3 * H), jnp.float32)],
)


def kernel(x, edge_index, batch, W1_0, b1_0, W2_0, b2_0, W1_1, b1_1, W2_1,
           b2_1, W1_2, b1_2, W2_2, b2_2, cW1, cb1, bn_gamma, bn_beta, cW2,
           cb2):
    src = edge_index[0]
    dst = edge_index[1]
    params = [(W1_0, b1_0, W2_0, b2_0), (W1_1, b1_1, W2_1, b2_1),
              (W1_2, b1_2, W2_2, b2_2)]

    h = x
    hs = []
    for (W1, b1, W2, b2) in params:
        agg = _sc_agg(h, src, dst)
        h = _mlp(h, agg, agg, W1, b1.reshape(1, H), W2, b2.reshape(1, H))
        hs.append(h)

    cW2p = jnp.zeros((2 * H, 128), jnp.float32).at[:, :NC].set(cW2)
    cb2p = jnp.zeros((1, 128), jnp.float32).at[0, :NC].set(cb2)
    out = _pool_cls(batch.reshape(N, 1), hs[0], hs[1], hs[2], cW1,
                    cb1.reshape(1, 2 * H), bn_gamma.reshape(1, 2 * H),
                    bn_beta.reshape(1, 2 * H), cW2p, cb2p)
    return out[:, :NC]


# speedup vs baseline: 4.0270x; 1.0333x over previous
import functools

import jax
import jax.numpy as jnp
import numpy as np
from jax import lax
from jax.experimental import pallas as pl
from jax.experimental.pallas import tpu as pltpu
from jax.experimental.pallas import tpu_sc as plsc

N = 10000
E = 320000
D = 128
H = 128
G = 64
NC = 2

NCORES = 2
NSUB = 16
NW = NCORES * NSUB
EPW = E // NW
CH = 128
NFULL = EPW // CH
TAIL = EPW - NFULL * CH
RPT = 640

_sc_mesh = plsc.VectorSubcoreMesh(core_axis_name="c", subcore_axis_name="s")


@functools.partial(
    pl.kernel,
    out_type=jax.ShapeDtypeStruct((2 * N, H), jnp.float32),
    mesh=_sc_mesh,
    scratch_types=[
        [pltpu.VMEM((CH,), jnp.int32) for _ in range(3)],
        [pltpu.VMEM((CH,), jnp.int32) for _ in range(6)],
        [pltpu.VMEM((CH, H), jnp.float32) for _ in range(3)],
        pltpu.VMEM((TAIL,), jnp.int32),
        pltpu.VMEM((TAIL,), jnp.int32),
        [pltpu.SemaphoreType.DMA for _ in range(3)],
        [pltpu.SemaphoreType.DMA for _ in range(3)],
        [pltpu.SemaphoreType.DMA for _ in range(2)],
        pltpu.SemaphoreType.DMA,
        pltpu.VMEM_SHARED((N, H), jnp.float32),
    ],
)
def _sc_agg(h_hbm, src_hbm, dst_hbm, out_hbm,
            sidxs, didxs, rows, sidx2, didx2, gsems, isems, ssems, tsem,
            agg_sh):
    rows0 = rows[0]
    cid = lax.axis_index("c")
    sid = lax.axis_index("s")
    wid = cid * NSUB + sid
    base = wid * EPW

    ZB = 128
    def _zrow(r, carry):
        for c8 in range(H // 16):
            rows0[r, pl.ds(c8 * 16, 16)] = jnp.zeros((16,), jnp.float32)
        return carry
    lax.fori_loop(0, ZB, _zrow, 0)
    row0 = sid * RPT

    @pl.when(sid < NSUB - 1)
    def _():
        for t in range(RPT // ZB):
            pltpu.sync_copy(rows0.at[pl.ds(0, ZB)],
                            agg_sh.at[pl.ds(row0 + t * ZB, ZB)])

    @pl.when(sid == NSUB - 1)
    def _():
        for t in range(3):
            pltpu.sync_copy(rows0.at[pl.ds(0, ZB)],
                            agg_sh.at[pl.ds(row0 + t * ZB, ZB)])
        last = N - (NSUB - 1) * RPT - 3 * ZB
        pltpu.sync_copy(rows0.at[pl.ds(0, last)],
                        agg_sh.at[pl.ds(row0 + 3 * ZB, last)])
    plsc.subcore_barrier()

    def _idx(j, r):
        off = base + j * CH
        pltpu.async_copy(src_hbm.at[pl.ds(off, CH)], sidxs[r % 3],
                         isems[r % 3])
        pltpu.async_copy(dst_hbm.at[pl.ds(off, CH)], didxs[r % 6],
                         isems[r % 3])

    def _iwait(j, r):
        off = base + j * CH
        pltpu.make_async_copy(src_hbm.at[pl.ds(off, CH)], sidxs[r % 3],
                              isems[r % 3]).wait()
        pltpu.make_async_copy(dst_hbm.at[pl.ds(off, CH)], didxs[r % 6],
                              isems[r % 3]).wait()

    def _gather(r):
        pltpu.async_copy(h_hbm.at[sidxs[r % 3]], rows[r % 3], gsems[r % 3])

    def _gwait(r):
        pltpu.make_async_copy(h_hbm.at[sidxs[r % 3]], rows[r % 3],
                              gsems[r % 3]).wait()

    def _sstart(r):
        pltpu.async_copy(rows[r % 3], agg_sh.at[didxs[r % 6]], ssems[r % 2],
                         add=True)

    def _swait(r):
        pltpu.make_async_copy(rows[r % 3], agg_sh.at[didxs[r % 6]],
                              ssems[r % 2]).wait()

    for j in range(3):
        _idx(j, j)
    for j in range(2):
        _iwait(j, j)
        _gather(j)

    def _slot(j, r):
        _gwait(r)
        _sstart(r)
        _idx(j + 3, r + 3)
        _swait(r + 5)
        _iwait(j + 2, r + 2)
        _gather(r + 2)

    _gwait(0); _sstart(0); _idx(3, 3); _iwait(2, 2); _gather(2)
    _gwait(1); _sstart(1); _idx(4, 4); _swait(0); _iwait(3, 3); _gather(3)

    def _six(t, carry):
        j0 = 6 * t + 2
        for k in range(6):
            _slot(j0 + k, 2 + k)
        return carry
    lax.fori_loop(0, 12, _six, 0)

    _gwait(74); _sstart(74); _idx(77, 77); _swait(73); _iwait(76, 76); _gather(76)
    _gwait(75); _sstart(75); _swait(74); _iwait(77, 77); _gather(77)
    _gwait(76); _sstart(76); _swait(75)
    off = base + NFULL * CH
    pltpu.sync_copy(src_hbm.at[pl.ds(off, TAIL)], sidx2)
    pltpu.sync_copy(dst_hbm.at[pl.ds(off, TAIL)], didx2)
    tail_dst = rows[0].at[pl.ds(0, TAIL)]
    pltpu.async_copy(h_hbm.at[sidx2], tail_dst, tsem)
    _gwait(77); _sstart(77); _swait(76)
    pltpu.make_async_copy(h_hbm.at[sidx2], tail_dst, tsem).wait()
    _swait(77)
    pltpu.sync_copy(tail_dst, agg_sh.at[didx2], add=True)

    plsc.subcore_barrier()

    @pl.when(sid < NSUB - 1)
    def _():
        pltpu.sync_copy(agg_sh.at[pl.ds(row0, RPT)],
                        out_hbm.at[pl.ds(cid * N + row0, RPT)])

    @pl.when(sid == NSUB - 1)
    def _():
        pltpu.sync_copy(agg_sh.at[pl.ds(row0, N - (NSUB - 1) * RPT)],
                        out_hbm.at[pl.ds(cid * N + row0, N - (NSUB - 1) * RPT)])


BR = 1000
NBLK = N // BR


def _mlp_body(h_ref, a0_ref, a1_ref, w1_ref, b1_ref, w2_ref, b2_ref, o_ref):
    z = h_ref[...] + a0_ref[...] + a1_ref[...]
    t = jnp.maximum(
        jnp.dot(z, w1_ref[...], preferred_element_type=jnp.float32) + b1_ref[...],
        0.0)
    o_ref[...] = jnp.maximum(
        jnp.dot(t, w2_ref[...], preferred_element_type=jnp.float32) + b2_ref[...],
        0.0)


_mlp = pl.pallas_call(
    _mlp_body,
    grid=(NBLK,),
    in_specs=[
        pl.BlockSpec((BR, H), lambda i: (i, 0)),
        pl.BlockSpec((BR, H), lambda i: (i, 0)),
        pl.BlockSpec((BR, H), lambda i: (NBLK + i, 0)),
        pl.BlockSpec((H, H), lambda i: (0, 0)),
        pl.BlockSpec((1, H), lambda i: (0, 0)),
        pl.BlockSpec((H, H), lambda i: (0, 0)),
        pl.BlockSpec((1, H), lambda i: (0, 0)),
    ],
    out_specs=pl.BlockSpec((BR, H), lambda i: (i, 0)),
    out_shape=jax.ShapeDtypeStruct((N, H), jnp.float32),
)

_BN_SCALE = float(1.0 / np.sqrt(1.0 + 1e-5))


def _mlp3_body(b_ref, h2_ref, a0_ref, a1_ref, w1_ref, b1_ref, w2_ref,
               b2_ref, h1_ref, cw1_ref, cb1_ref, g_ref, be_ref, cw2_ref,
               cb2_ref, o_ref, acc_ref):
    i = pl.program_id(0)

    @pl.when(i == 0)
    def _():
        acc_ref[...] = jnp.zeros_like(acc_ref)

    z = h2_ref[...] + a0_ref[...] + a1_ref[...]
    t = jnp.maximum(
        jnp.dot(z, w1_ref[...], preferred_element_type=jnp.float32) + b1_ref[...],
        0.0)
    h3 = jnp.maximum(
        jnp.dot(t, w2_ref[...], preferred_element_type=jnp.float32) + b2_ref[...],
        0.0)
    oh = (b_ref[...] == lax.broadcasted_iota(jnp.int32, (1, G), 1)
          ).astype(jnp.float32)
    hcat = jnp.concatenate([h1_ref[...], h2_ref[...], h3], axis=1)
    acc_ref[...] += jnp.dot(oh.T, hcat, preferred_element_type=jnp.float32)

    @pl.when(i == pl.num_programs(0) - 1)
    def _():
        zc = jnp.dot(acc_ref[...], cw1_ref[...],
                     preferred_element_type=jnp.float32) + cb1_ref[...]
        zc = zc * _BN_SCALE * g_ref[...] + be_ref[...]
        zc = jnp.maximum(zc, 0.0)
        o_ref[...] = jnp.dot(zc, cw2_ref[...],
                             preferred_element_type=jnp.float32) + cb2_ref[...]


_mlp3_pool = pl.pallas_call(
    _mlp3_body,
    grid=(NBLK,),
    in_specs=[
        pl.BlockSpec((BR, 1), lambda i: (i, 0)),
        pl.BlockSpec((BR, H), lambda i: (i, 0)),
        pl.BlockSpec((BR, H), lambda i: (i, 0)),
        pl.BlockSpec((BR, H), lambda i: (NBLK + i, 0)),
        pl.BlockSpec((H, H), lambda i: (0, 0)),
        pl.BlockSpec((1, H), lambda i: (0, 0)),
        pl.BlockSpec((H, H), lambda i: (0, 0)),
        pl.BlockSpec((1, H), lambda i: (0, 0)),
        pl.BlockSpec((BR, H), lambda i: (i, 0)),
        pl.BlockSpec((3 * H, 2 * H), lambda i: (0, 0)),
        pl.BlockSpec((1, 2 * H), lambda i: (0, 0)),
        pl.BlockSpec((1, 2 * H), lambda i: (0, 0)),
        pl.BlockSpec((1, 2 * H), lambda i: (0, 0)),
        pl.BlockSpec((2 * H, 128), lambda i: (0, 0)),
        pl.BlockSpec((1, 128), lambda i: (0, 0)),
    ],
    out_specs=pl.BlockSpec((G, 128), lambda i: (0, 0)),
    out_shape=jax.ShapeDtypeStruct((G, 128), jnp.float32),
    scratch_shapes=[pltpu.VMEM((G, 3 * H), jnp.float32)],
)


def kernel(x, edge_index, batch, W1_0, b1_0, W2_0, b2_0, W1_1, b1_1, W2_1,
           b2_1, W1_2, b1_2, W2_2, b2_2, cW1, cb1, bn_gamma, bn_beta, cW2,
           cb2):
    src = edge_index[0]
    dst = edge_index[1]
    params = [(W1_0, b1_0, W2_0, b2_0), (W1_1, b1_1, W2_1, b2_1),
              (W1_2, b1_2, W2_2, b2_2)]

    h = x
    hs = []
    for (W1, b1, W2, b2) in params[:2]:
        agg = _sc_agg(h, src, dst)
        h = _mlp(h, agg, agg, W1, b1.reshape(1, H), W2, b2.reshape(1, H))
        hs.append(h)

    cW2p = jnp.zeros((2 * H, 128), jnp.float32).at[:, :NC].set(cW2)
    cb2p = jnp.zeros((1, 128), jnp.float32).at[0, :NC].set(cb2)
    agg = _sc_agg(h, src, dst)
    out = _mlp3_pool(batch.reshape(N, 1), h, agg, agg, W1_2,
                     b1_2.reshape(1, H), W2_2, b2_2.reshape(1, H), hs[0],
                     cW1, cb1.reshape(1, 2 * H), bn_gamma.reshape(1, 2 * H),
                     bn_beta.reshape(1, 2 * H), cW2p, cb2p)
    return out[:, :NC]


# speedup vs baseline: 4.0488x; 1.0054x over previous
import functools

import jax
import jax.numpy as jnp
import numpy as np
from jax import lax
from jax.experimental import pallas as pl
from jax.experimental.pallas import tpu as pltpu
from jax.experimental.pallas import tpu_sc as plsc

N = 10000
E = 320000
D = 128
H = 128
G = 64
NC = 2

NCORES = 2
NSUB = 16
NW = NCORES * NSUB
EPW = E // NW
CH = 128
NFULL = EPW // CH
TAIL = EPW - NFULL * CH
RPT = 640

_sc_mesh = plsc.VectorSubcoreMesh(core_axis_name="c", subcore_axis_name="s")


@functools.partial(
    pl.kernel,
    out_type=jax.ShapeDtypeStruct((2 * N, H), jnp.float32),
    mesh=_sc_mesh,
    scratch_types=[
        [pltpu.VMEM((CH,), jnp.int32) for _ in range(3)],
        [pltpu.VMEM((CH,), jnp.int32) for _ in range(6)],
        [pltpu.VMEM((CH, H), jnp.float32) for _ in range(3)],
        pltpu.VMEM((TAIL,), jnp.int32),
        pltpu.VMEM((TAIL,), jnp.int32),
        [pltpu.SemaphoreType.DMA for _ in range(3)],
        [pltpu.SemaphoreType.DMA for _ in range(3)],
        [pltpu.SemaphoreType.DMA for _ in range(2)],
        pltpu.SemaphoreType.DMA,
        pltpu.VMEM_SHARED((N, H), jnp.float32),
    ],
)
def _sc_agg(h_hbm, src_hbm, dst_hbm, out_hbm,
            sidxs, didxs, rows, sidx2, didx2, gsems, isems, ssems, tsem,
            agg_sh):
    rows0 = rows[0]
    cid = lax.axis_index("c")
    sid = lax.axis_index("s")
    wid = cid * NSUB + sid
    base = wid * EPW

    ZB = 128
    zbuf = rows[2]
    def _zrow(r, carry):
        for c8 in range(H // 16):
            zbuf[r, pl.ds(c8 * 16, 16)] = jnp.zeros((16,), jnp.float32)
        return carry
    lax.fori_loop(0, ZB, _zrow, 0)
    row0 = sid * RPT

    def _idx(j, r):
        off = base + j * CH
        pltpu.async_copy(src_hbm.at[pl.ds(off, CH)], sidxs[r % 3],
                         isems[r % 3])
        pltpu.async_copy(dst_hbm.at[pl.ds(off, CH)], didxs[r % 6],
                         isems[r % 3])

    def _iwait(j, r):
        off = base + j * CH
        pltpu.make_async_copy(src_hbm.at[pl.ds(off, CH)], sidxs[r % 3],
                              isems[r % 3]).wait()
        pltpu.make_async_copy(dst_hbm.at[pl.ds(off, CH)], didxs[r % 6],
                              isems[r % 3]).wait()

    def _gather(r):
        pltpu.async_copy(h_hbm.at[sidxs[r % 3]], rows[r % 3], gsems[r % 3])

    def _gwait(r):
        pltpu.make_async_copy(h_hbm.at[sidxs[r % 3]], rows[r % 3],
                              gsems[r % 3]).wait()

    def _sstart(r):
        pltpu.async_copy(rows[r % 3], agg_sh.at[didxs[r % 6]], ssems[r % 2],
                         add=True)

    def _swait(r):
        pltpu.make_async_copy(rows[r % 3], agg_sh.at[didxs[r % 6]],
                              ssems[r % 2]).wait()

    for j in range(3):
        _idx(j, j)

    @pl.when(sid < NSUB - 1)
    def _():
        for t in range(RPT // ZB):
            pltpu.sync_copy(zbuf.at[pl.ds(0, ZB)],
                            agg_sh.at[pl.ds(row0 + t * ZB, ZB)])

    @pl.when(sid == NSUB - 1)
    def _():
        for t in range(3):
            pltpu.sync_copy(zbuf.at[pl.ds(0, ZB)],
                            agg_sh.at[pl.ds(row0 + t * ZB, ZB)])
        last = N - (NSUB - 1) * RPT - 3 * ZB
        pltpu.sync_copy(zbuf.at[pl.ds(0, last)],
                        agg_sh.at[pl.ds(row0 + 3 * ZB, last)])

    for j in range(2):
        _iwait(j, j)
        _gather(j)
    plsc.subcore_barrier()

    def _slot(j, r):
        _gwait(r)
        _sstart(r)
        _idx(j + 3, r + 3)
        _swait(r + 5)
        _iwait(j + 2, r + 2)
        _gather(r + 2)

    _gwait(0); _sstart(0); _idx(3, 3); _iwait(2, 2); _gather(2)
    _gwait(1); _sstart(1); _idx(4, 4); _swait(0); _iwait(3, 3); _gather(3)

    def _six(t, carry):
        j0 = 6 * t + 2
        for k in range(6):
            _slot(j0 + k, 2 + k)
        return carry
    lax.fori_loop(0, 12, _six, 0)

    _gwait(74); _sstart(74); _idx(77, 77); _swait(73); _iwait(76, 76); _gather(76)
    _gwait(75); _sstart(75); _swait(74); _iwait(77, 77); _gather(77)
    _gwait(76); _sstart(76); _swait(75)
    off = base + NFULL * CH
    pltpu.sync_copy(src_hbm.at[pl.ds(off, TAIL)], sidx2)
    pltpu.sync_copy(dst_hbm.at[pl.ds(off, TAIL)], didx2)
    tail_dst = rows[0].at[pl.ds(0, TAIL)]
    pltpu.async_copy(h_hbm.at[sidx2], tail_dst, tsem)
    _gwait(77); _sstart(77); _swait(76)
    pltpu.make_async_copy(h_hbm.at[sidx2], tail_dst, tsem).wait()
    _swait(77)
    pltpu.sync_copy(tail_dst, agg_sh.at[didx2], add=True)

    plsc.subcore_barrier()

    @pl.when(sid < NSUB - 1)
    def _():
        pltpu.sync_copy(agg_sh.at[pl.ds(row0, RPT)],
                        out_hbm.at[pl.ds(cid * N + row0, RPT)])

    @pl.when(sid == NSUB - 1)
    def _():
        pltpu.sync_copy(agg_sh.at[pl.ds(row0, N - (NSUB - 1) * RPT)],
                        out_hbm.at[pl.ds(cid * N + row0, N - (NSUB - 1) * RPT)])


BR = 1000
NBLK = N // BR


def _mlp_body(h_ref, a0_ref, a1_ref, w1_ref, b1_ref, w2_ref, b2_ref, o_ref):
    z = h_ref[...] + a0_ref[...] + a1_ref[...]
    t = jnp.maximum(
        jnp.dot(z, w1_ref[...], preferred_element_type=jnp.float32) + b1_ref[...],
        0.0)
    o_ref[...] = jnp.maximum(
        jnp.dot(t, w2_ref[...], preferred_element_type=jnp.float32) + b2_ref[...],
        0.0)


_mlp = pl.pallas_call(
    _mlp_body,
    grid=(NBLK,),
    in_specs=[
        pl.BlockSpec((BR, H), lambda i: (i, 0)),
        pl.BlockSpec((BR, H), lambda i: (i, 0)),
        pl.BlockSpec((BR, H), lambda i: (NBLK + i, 0)),
        pl.BlockSpec((H, H), lambda i: (0, 0)),
        pl.BlockSpec((1, H), lambda i: (0, 0)),
        pl.BlockSpec((H, H), lambda i: (0, 0)),
        pl.BlockSpec((1, H), lambda i: (0, 0)),
    ],
    out_specs=pl.BlockSpec((BR, H), lambda i: (i, 0)),
    out_shape=jax.ShapeDtypeStruct((N, H), jnp.float32),
)

_BN_SCALE = float(1.0 / np.sqrt(1.0 + 1e-5))


def _mlp3_body(b_ref, h2_ref, a0_ref, a1_ref, w1_ref, b1_ref, w2_ref,
               b2_ref, h1_ref, cw1_ref, cb1_ref, g_ref, be_ref, cw2_ref,
               cb2_ref, o_ref, acc_ref):
    i = pl.program_id(0)

    @pl.when(i == 0)
    def _():
        acc_ref[...] = jnp.zeros_like(acc_ref)

    z = h2_ref[...] + a0_ref[...] + a1_ref[...]
    t = jnp.maximum(
        jnp.dot(z, w1_ref[...], preferred_element_type=jnp.float32) + b1_ref[...],
        0.0)
    h3 = jnp.maximum(
        jnp.dot(t, w2_ref[...], preferred_element_type=jnp.float32) + b2_ref[...],
        0.0)
    oh = (b_ref[...] == lax.broadcasted_iota(jnp.int32, (1, G), 1)
          ).astype(jnp.float32)
    hcat = jnp.concatenate([h1_ref[...], h2_ref[...], h3], axis=1)
    acc_ref[...] += jnp.dot(oh.T, hcat, preferred_element_type=jnp.float32)

    @pl.when(i == pl.num_programs(0) - 1)
    def _():
        zc = jnp.dot(acc_ref[...], cw1_ref[...],
                     preferred_element_type=jnp.float32) + cb1_ref[...]
        zc = zc * _BN_SCALE * g_ref[...] + be_ref[...]
        zc = jnp.maximum(zc, 0.0)
        o_ref[...] = jnp.dot(zc, cw2_ref[...],
                             preferred_element_type=jnp.float32) + cb2_ref[...]


_mlp3_pool = pl.pallas_call(
    _mlp3_body,
    grid=(NBLK,),
    in_specs=[
        pl.BlockSpec((BR, 1), lambda i: (i, 0)),
        pl.BlockSpec((BR, H), lambda i: (i, 0)),
        pl.BlockSpec((BR, H), lambda i: (i, 0)),
        pl.BlockSpec((BR, H), lambda i: (NBLK + i, 0)),
        pl.BlockSpec((H, H), lambda i: (0, 0)),
        pl.BlockSpec((1, H), lambda i: (0, 0)),
        pl.BlockSpec((H, H), lambda i: (0, 0)),
        pl.BlockSpec((1, H), lambda i: (0, 0)),
        pl.BlockSpec((BR, H), lambda i: (i, 0)),
        pl.BlockSpec((3 * H, 2 * H), lambda i: (0, 0)),
        pl.BlockSpec((1, 2 * H), lambda i: (0, 0)),
        pl.BlockSpec((1, 2 * H), lambda i: (0, 0)),
        pl.BlockSpec((1, 2 * H), lambda i: (0, 0)),
        pl.BlockSpec((2 * H, 128), lambda i: (0, 0)),
        pl.BlockSpec((1, 128), lambda i: (0, 0)),
    ],
    out_specs=pl.BlockSpec((G, 128), lambda i: (0, 0)),
    out_shape=jax.ShapeDtypeStruct((G, 128), jnp.float32),
    scratch_shapes=[pltpu.VMEM((G, 3 * H), jnp.float32)],
)


def kernel(x, edge_index, batch, W1_0, b1_0, W2_0, b2_0, W1_1, b1_1, W2_1,
           b2_1, W1_2, b1_2, W2_2, b2_2, cW1, cb1, bn_gamma, bn_beta, cW2,
           cb2):
    src = edge_index[0]
    dst = edge_index[1]
    params = [(W1_0, b1_0, W2_0, b2_0), (W1_1, b1_1, W2_1, b2_1),
              (W1_2, b1_2, W2_2, b2_2)]

    h = x
    hs = []
    for (W1, b1, W2, b2) in params[:2]:
        agg = _sc_agg(h, src, dst)
        h = _mlp(h, agg, agg, W1, b1.reshape(1, H), W2, b2.reshape(1, H))
        hs.append(h)

    cW2p = jnp.zeros((2 * H, 128), jnp.float32).at[:, :NC].set(cW2)
    cb2p = jnp.zeros((1, 128), jnp.float32).at[0, :NC].set(cb2)
    agg = _sc_agg(h, src, dst)
    out = _mlp3_pool(batch.reshape(N, 1), h, agg, agg, W1_2,
                     b1_2.reshape(1, H), W2_2, b2_2.reshape(1, H), hs[0],
                     cW1, cb1.reshape(1, 2 * H), bn_gamma.reshape(1, 2 * H),
                     bn_beta.reshape(1, 2 * H), cW2p, cb2p)
    return out[:, :NC]


# speedup vs baseline: 4.0499x; 1.0003x over previous
import functools

import jax
import jax.numpy as jnp
import numpy as np
from jax import lax
from jax.experimental import pallas as pl
from jax.experimental.pallas import tpu as pltpu
from jax.experimental.pallas import tpu_sc as plsc

N = 10000
E = 320000
D = 128
H = 128
G = 64
NC = 2

NCORES = 2
NSUB = 16
NW = NCORES * NSUB
EPW = E // NW
CH = 128
NFULL = EPW // CH
TAIL = EPW - NFULL * CH
RPT = 640

_sc_mesh = plsc.VectorSubcoreMesh(core_axis_name="c", subcore_axis_name="s")


@functools.partial(
    pl.kernel,
    out_type=jax.ShapeDtypeStruct((2 * N, H), jnp.float32),
    mesh=_sc_mesh,
    scratch_types=[
        [pltpu.VMEM((CH,), jnp.int32) for _ in range(3)],
        [pltpu.VMEM((CH,), jnp.int32) for _ in range(6)],
        [pltpu.VMEM((CH, H), jnp.float32) for _ in range(3)],
        pltpu.VMEM((TAIL,), jnp.int32),
        pltpu.VMEM((TAIL,), jnp.int32),
        [pltpu.SemaphoreType.DMA for _ in range(3)],
        [pltpu.SemaphoreType.DMA for _ in range(3)],
        [pltpu.SemaphoreType.DMA for _ in range(2)],
        pltpu.SemaphoreType.DMA,
        pltpu.VMEM_SHARED((N, H), jnp.float32),
    ],
)
def _sc_agg(h_hbm, src_hbm, dst_hbm, out_hbm,
            sidxs, didxs, rows, sidx2, didx2, gsems, isems, ssems, tsem,
            agg_sh):
    rows0 = rows[0]
    cid = lax.axis_index("c")
    sid = lax.axis_index("s")
    wid = cid * NSUB + sid
    base = wid * EPW

    ZB = 128
    zbuf = rows[2]
    def _zrow(r, carry):
        for c8 in range(H // 16):
            zbuf[r, pl.ds(c8 * 16, 16)] = jnp.zeros((16,), jnp.float32)
        return carry
    lax.fori_loop(0, ZB, _zrow, 0)
    row0 = sid * RPT

    def _idx(j, r):
        off = base + j * CH
        pltpu.async_copy(src_hbm.at[pl.ds(off, CH)], sidxs[r % 3],
                         isems[r % 3])
        pltpu.async_copy(dst_hbm.at[pl.ds(off, CH)], didxs[r % 6],
                         isems[r % 3])

    def _iwait(j, r):
        off = base + j * CH
        pltpu.make_async_copy(src_hbm.at[pl.ds(off, CH)], sidxs[r % 3],
                              isems[r % 3]).wait()
        pltpu.make_async_copy(dst_hbm.at[pl.ds(off, CH)], didxs[r % 6],
                              isems[r % 3]).wait()

    def _gather(r):
        pltpu.async_copy(h_hbm.at[sidxs[r % 3]], rows[r % 3], gsems[r % 3])

    def _gwait(r):
        pltpu.make_async_copy(h_hbm.at[sidxs[r % 3]], rows[r % 3],
                              gsems[r % 3]).wait()

    def _sstart(r):
        pltpu.async_copy(rows[r % 3], agg_sh.at[didxs[r % 6]], ssems[r % 2],
                         add=True)

    def _swait(r):
        pltpu.make_async_copy(rows[r % 3], agg_sh.at[didxs[r % 6]],
                              ssems[r % 2]).wait()

    for j in range(3):
        _idx(j, j)
    toff = base + NFULL * CH
    pltpu.async_copy(src_hbm.at[pl.ds(toff, TAIL)], sidx2, tsem)
    pltpu.async_copy(dst_hbm.at[pl.ds(toff, TAIL)], didx2, tsem)

    @pl.when(sid < NSUB - 1)
    def _():
        for t in range(RPT // ZB):
            pltpu.sync_copy(zbuf.at[pl.ds(0, ZB)],
                            agg_sh.at[pl.ds(row0 + t * ZB, ZB)])

    @pl.when(sid == NSUB - 1)
    def _():
        for t in range(3):
            pltpu.sync_copy(zbuf.at[pl.ds(0, ZB)],
                            agg_sh.at[pl.ds(row0 + t * ZB, ZB)])
        last = N - (NSUB - 1) * RPT - 3 * ZB
        pltpu.sync_copy(zbuf.at[pl.ds(0, last)],
                        agg_sh.at[pl.ds(row0 + 3 * ZB, last)])

    for j in range(2):
        _iwait(j, j)
        _gather(j)
    plsc.subcore_barrier()

    def _slot(j, r):
        _gwait(r)
        _sstart(r)
        _idx(j + 3, r + 3)
        _swait(r + 5)
        _iwait(j + 2, r + 2)
        _gather(r + 2)

    _gwait(0); _sstart(0); _idx(3, 3); _iwait(2, 2); _gather(2)
    _gwait(1); _sstart(1); _idx(4, 4); _swait(0); _iwait(3, 3); _gather(3)

    def _six(t, carry):
        j0 = 6 * t + 2
        for k in range(6):
            _slot(j0 + k, 2 + k)
        return carry
    lax.fori_loop(0, 12, _six, 0)

    _gwait(74); _sstart(74); _idx(77, 77); _swait(73); _iwait(76, 76); _gather(76)
    _gwait(75); _sstart(75); _swait(74); _iwait(77, 77); _gather(77)
    _gwait(76); _sstart(76); _swait(75)
    pltpu.make_async_copy(src_hbm.at[pl.ds(toff, TAIL)], sidx2, tsem).wait()
    pltpu.make_async_copy(dst_hbm.at[pl.ds(toff, TAIL)], didx2, tsem).wait()
    tail_dst = rows[0].at[pl.ds(0, TAIL)]
    pltpu.async_copy(h_hbm.at[sidx2], tail_dst, tsem)
    _gwait(77); _sstart(77); _swait(76)
    pltpu.make_async_copy(h_hbm.at[sidx2], tail_dst, tsem).wait()
    _swait(77)
    pltpu.sync_copy(tail_dst, agg_sh.at[didx2], add=True)

    plsc.subcore_barrier()

    @pl.when(sid < NSUB - 1)
    def _():
        pltpu.sync_copy(agg_sh.at[pl.ds(row0, RPT)],
                        out_hbm.at[pl.ds(cid * N + row0, RPT)])

    @pl.when(sid == NSUB - 1)
    def _():
        pltpu.sync_copy(agg_sh.at[pl.ds(row0, N - (NSUB - 1) * RPT)],
                        out_hbm.at[pl.ds(cid * N + row0, N - (NSUB - 1) * RPT)])


BR = 1000
NBLK = N // BR


def _mlp_body(h_ref, a0_ref, a1_ref, w1_ref, b1_ref, w2_ref, b2_ref, o_ref):
    z = h_ref[...] + a0_ref[...] + a1_ref[...]
    t = jnp.maximum(
        jnp.dot(z, w1_ref[...], preferred_element_type=jnp.float32) + b1_ref[...],
        0.0)
    o_ref[...] = jnp.maximum(
        jnp.dot(t, w2_ref[...], preferred_element_type=jnp.float32) + b2_ref[...],
        0.0)


_mlp = pl.pallas_call(
    _mlp_body,
    grid=(NBLK,),
    in_specs=[
        pl.BlockSpec((BR, H), lambda i: (i, 0)),
        pl.BlockSpec((BR, H), lambda i: (i, 0)),
        pl.BlockSpec((BR, H), lambda i: (NBLK + i, 0)),
        pl.BlockSpec((H, H), lambda i: (0, 0)),
        pl.BlockSpec((1, H), lambda i: (0, 0)),
        pl.BlockSpec((H, H), lambda i: (0, 0)),
        pl.BlockSpec((1, H), lambda i: (0, 0)),
    ],
    out_specs=pl.BlockSpec((BR, H), lambda i: (i, 0)),
    out_shape=jax.ShapeDtypeStruct((N, H), jnp.float32),
)

_BN_SCALE = float(1.0 / np.sqrt(1.0 + 1e-5))


def _mlp3_body(b_ref, h2_ref, a0_ref, a1_ref, w1_ref, b1_ref, w2_ref,
               b2_ref, h1_ref, cw1_ref, cb1_ref, g_ref, be_ref, cw2_ref,
               cb2_ref, o_ref, acc_ref):
    i = pl.program_id(0)

    @pl.when(i == 0)
    def _():
        acc_ref[...] = jnp.zeros_like(acc_ref)

    z = h2_ref[...] + a0_ref[...] + a1_ref[...]
    t = jnp.maximum(
        jnp.dot(z, w1_ref[...], preferred_element_type=jnp.float32) + b1_ref[...],
        0.0)
    h3 = jnp.maximum(
        jnp.dot(t, w2_ref[...], preferred_element_type=jnp.float32) + b2_ref[...],
        0.0)
    oh = (b_ref[...] == lax.broadcasted_iota(jnp.int32, (1, G), 1)
          ).astype(jnp.float32)
    hcat = jnp.concatenate([h1_ref[...], h2_ref[...], h3], axis=1)
    acc_ref[...] += jnp.dot(oh.T, hcat, preferred_element_type=jnp.float32)

    @pl.when(i == pl.num_programs(0) - 1)
    def _():
        zc = jnp.dot(acc_ref[...], cw1_ref[...],
                     preferred_element_type=jnp.float32) + cb1_ref[...]
        zc = zc * _BN_SCALE * g_ref[...] + be_ref[...]
        zc = jnp.maximum(zc, 0.0)
        o_ref[...] = jnp.dot(zc, cw2_ref[...],
                             preferred_element_type=jnp.float32) + cb2_ref[...]


_mlp3_pool = pl.pallas_call(
    _mlp3_body,
    grid=(NBLK,),
    in_specs=[
        pl.BlockSpec((BR, 1), lambda i: (i, 0)),
        pl.BlockSpec((BR, H), lambda i: (i, 0)),
        pl.BlockSpec((BR, H), lambda i: (i, 0)),
        pl.BlockSpec((BR, H), lambda i: (NBLK + i, 0)),
        pl.BlockSpec((H, H), lambda i: (0, 0)),
        pl.BlockSpec((1, H), lambda i: (0, 0)),
        pl.BlockSpec((H, H), lambda i: (0, 0)),
        pl.BlockSpec((1, H), lambda i: (0, 0)),
        pl.BlockSpec((BR, H), lambda i: (i, 0)),
        pl.BlockSpec((3 * H, 2 * H), lambda i: (0, 0)),
        pl.BlockSpec((1, 2 * H), lambda i: (0, 0)),
        pl.BlockSpec((1, 2 * H), lambda i: (0, 0)),
        pl.BlockSpec((1, 2 * H), lambda i: (0, 0)),
        pl.BlockSpec((2 * H, NC), lambda i: (0, 0)),
        pl.BlockSpec((1, NC), lambda i: (0, 0)),
    ],
    out_specs=pl.BlockSpec((G, NC), lambda i: (0, 0)),
    out_shape=jax.ShapeDtypeStruct((G, NC), jnp.float32),
    scratch_shapes=[pltpu.VMEM((G, 3 * H), jnp.float32)],
)


def kernel(x, edge_index, batch, W1_0, b1_0, W2_0, b2_0, W1_1, b1_1, W2_1,
           b2_1, W1_2, b1_2, W2_2, b2_2, cW1, cb1, bn_gamma, bn_beta, cW2,
           cb2):
    src = edge_index[0]
    dst = edge_index[1]
    params = [(W1_0, b1_0, W2_0, b2_0), (W1_1, b1_1, W2_1, b2_1),
              (W1_2, b1_2, W2_2, b2_2)]

    h = x
    hs = []
    for (W1, b1, W2, b2) in params[:2]:
        agg = _sc_agg(h, src, dst)
        h = _mlp(h, agg, agg, W1, b1.reshape(1, H), W2, b2.reshape(1, H))
        hs.append(h)

    agg = _sc_agg(h, src, dst)
    return _mlp3_pool(batch.reshape(N, 1), h, agg, agg, W1_2,
                      b1_2.reshape(1, H), W2_2, b2_2.reshape(1, H), hs[0],
                      cW1, cb1.reshape(1, 2 * H), bn_gamma.reshape(1, 2 * H),
                      bn_beta.reshape(1, 2 * H), cW2, cb2.reshape(1, NC))
